# Initial kernel scaffold; baseline (speedup 1.0000x reference)
import jax
import jax.numpy as jnp
from jax.experimental import pallas as pl


def kernel(anchor_x, anchor_edge_index, anchor_edge_attr, anchor_batch, positive_x, positive_edge_index, positive_edge_attr, positive_batch, negative_x, negative_edge_index, negative_edge_attr, negative_batch, W_enc, b_enc, eps, W_edge, b_edge, W1, b1, W2, b2, W_fc, b_fc):
    raise NotImplementedError("write your pallas kernel here")


# speedup vs baseline: 4.5076x; 4.5076x over previous
import functools

import jax
import jax.numpy as jnp
from jax import lax
from jax.experimental import pallas as pl
from jax.experimental.pallas import tpu as pltpu
from jax.experimental.pallas import tpu_sc as plsc

N = 50000
E = 800000
H = 64
OUT = 64
G = 256

NC = 2
NS = 16
LN = 128
E2 = 819200
K = 512
NB = (E2 // NS) // K
KR = K // LN
CHUNK = 128
NCH = K // CHUNK
NGRP = K // 16
NPAD = 8
XP = 50048
RPT = 3128
RLAST = N - 15 * RPT
CH = 32


def _edge_kernel(xa_hbm, xb_hbm, xc_hbm, src_hbm, dst_hbm, ea0_hbm, ea1_hbm,
                 par_hbm, zer_hbm, aggr_hbm,
                 xs_v, src_v, dst_v, ea0_v, ea1_v, m_v, p_v, aggr_sh, sem):
    x_hbms = (xa_hbm, xb_hbm, xc_hbm)
    c = lax.axis_index("c")
    t = lax.axis_index("s")
    pltpu.sync_copy(par_hbm.at[c], p_v)
    u0 = p_v[0, 0:16]
    u1 = p_v[0, 16:32]
    v0 = p_v[1, 0:16]
    v1 = p_v[1, 16:32]
    w0 = p_v[2, 0:16]
    w1 = p_v[2, 16:32]
    b0 = p_v[3, 0:16]
    b1 = p_v[3, 16:32]

    for g in range(3):
        xg_hbm = x_hbms[g]

        @pl.when(t < NS - 1)
        def _():
            pltpu.sync_copy(zer_hbm, aggr_sh.at[pl.ds(t * RPT, RPT)])

        @pl.when(t == NS - 1)
        def _():
            pltpu.sync_copy(zer_hbm.at[pl.ds(0, RLAST + NPAD)],
                            aggr_sh.at[pl.ds(15 * RPT, RLAST + NPAD)])

        plsc.subcore_barrier()

        def block_body(b, carry):
            row = t * (NB * KR) + b * KR
            pltpu.sync_copy(src_hbm.at[g, pl.ds(row, KR), :], src_v)
            pltpu.sync_copy(ea0_hbm.at[g, pl.ds(row, KR), :], ea0_v)
            pltpu.sync_copy(ea1_hbm.at[g, pl.ds(row, KR), :], ea1_v)
            pltpu.sync_copy(dst_hbm.at[g, pl.ds(row, KR), :], dst_v)
            copies = [pltpu.make_async_copy(xg_hbm.at[src_v.at[r]],
                                            xs_v.at[r], sem)
                      for r in range(KR)]
            for cp in copies:
                cp.start()
            for cp in copies:
                cp.wait()

            def group_body(q, carry2):
                r = q >> 3
                o = (q & 7) * 16
                xs = xs_v[r, pl.ds(o, 16)]
                a0 = ea0_v[r, pl.ds(o, 16)]
                a1 = ea1_v[r, pl.ds(o, 16)]
                jb = q * 16
                for e in range(16):
                    xe = jnp.full((16,), xs[e])
                    ae = jnp.full((16,), a0[e])
                    ce = jnp.full((16,), a1[e])
                    m_v[jb + e, 0:16] = jnp.maximum(
                        xe * u0 + ae * v0 + ce * w0 + b0, 0.0)
                    m_v[jb + e, 16:32] = jnp.maximum(
                        xe * u1 + ae * v1 + ce * w1 + b1, 0.0)
                return carry2

            lax.fori_loop(0, NGRP, group_body, 0)
            for i in range(NCH):
                pltpu.sync_copy(m_v.at[pl.ds(i * CHUNK, CHUNK)],
                                aggr_sh.at[dst_v.at[i]], add=True)
            return carry

        lax.fori_loop(0, NB, block_body, 0)
        plsc.subcore_barrier()

        @pl.when(t < NS - 1)
        def _():
            pltpu.sync_copy(aggr_sh.at[pl.ds(t * RPT, RPT)],
                            aggr_hbm.at[g, c, pl.ds(t * RPT, RPT), :])

        @pl.when(t == NS - 1)
        def _():
            pltpu.sync_copy(aggr_sh.at[pl.ds(15 * RPT, RLAST)],
                            aggr_hbm.at[g, c, pl.ds(15 * RPT, RLAST), :])

        plsc.subcore_barrier()


def _edge_stage(xa, xb, xc, src3, dst3, ea0, ea1, params, zeros):
    mesh = plsc.VectorSubcoreMesh(core_axis_name="c", subcore_axis_name="s")
    return pl.kernel(
        _edge_kernel,
        mesh=mesh,
        compiler_params=pltpu.CompilerParams(use_tc_tiling_on_sc=False),
        out_type=jax.ShapeDtypeStruct((3, NC, N, CH), jnp.float32),
        scratch_types=[
            pltpu.VMEM((KR, LN), jnp.float32),
            pltpu.VMEM((KR, LN), jnp.int32),
            pltpu.VMEM((KR, LN), jnp.int32),
            pltpu.VMEM((KR, LN), jnp.float32),
            pltpu.VMEM((KR, LN), jnp.float32),
            pltpu.VMEM((K, CH), jnp.float32),
            pltpu.VMEM((4, CH), jnp.float32),
            pltpu.VMEM_SHARED((N + NPAD, CH), jnp.float32),
            pltpu.SemaphoreType.DMA,
        ],
    )(xa, xb, xc, src3, dst3, ea0, ea1, params, zeros)


BN = 2000
NBLK = N // BN


def _dense_kernel(x_ref, alo_ref, ahi_ref, batch_ref, scale_ref, wenc_ref,
                  benc_ref, w1_ref, b1_ref, w2_ref, b2_ref, wfc_ref, bfc_ref,
                  out_ref, pooled_scr):
    nb = pl.program_id(1)

    @pl.when(nb == 0)
    def _():
        pooled_scr[...] = jnp.zeros_like(pooled_scr)

    xcol = x_ref[0]
    h = xcol * wenc_ref[...] + benc_ref[...]
    aggr = jnp.concatenate([alo_ref[0, 0], ahi_ref[0, 0]], axis=-1)
    z = scale_ref[0, 0] * h + aggr
    z = jnp.maximum(jnp.dot(z, w1_ref[...], preferred_element_type=jnp.float32)
                    + b1_ref[...], 0.0)
    z = jnp.maximum(jnp.dot(z, w2_ref[...], preferred_element_type=jnp.float32)
                    + b2_ref[...], 0.0)
    seg = batch_ref[0, 0, :]
    onehot = (lax.broadcasted_iota(jnp.int32, (G, BN), 0) ==
              seg[None, :]).astype(jnp.float32)
    pooled_scr[...] += jnp.dot(onehot, z, preferred_element_type=jnp.float32)

    @pl.when(nb == NBLK - 1)
    def _():
        out_ref[0] = (jnp.dot(pooled_scr[...], wfc_ref[...],
                              preferred_element_type=jnp.float32)
                      + bfc_ref[...])


def _dense_stage(x3, aggr, batch3, scale, W_enc, b_enc, W1, b1, W2, b2,
                 W_fc, b_fc):
    return pl.pallas_call(
        _dense_kernel,
        grid=(3, NBLK),
        in_specs=[
            pl.BlockSpec((1, BN, 1), lambda g, nb: (g, nb, 0)),
            pl.BlockSpec((1, 1, BN, CH), lambda g, nb: (g, 0, nb, 0)),
            pl.BlockSpec((1, 1, BN, CH), lambda g, nb: (g, 1, nb, 0)),
            pl.BlockSpec((1, 1, BN), lambda g, nb: (g * NBLK + nb, 0, 0)),
            pl.BlockSpec((1, 1), lambda g, nb: (0, 0)),
            pl.BlockSpec((1, H), lambda g, nb: (0, 0)),
            pl.BlockSpec((1, H), lambda g, nb: (0, 0)),
            pl.BlockSpec((H, H), lambda g, nb: (0, 0)),
            pl.BlockSpec((1, H), lambda g, nb: (0, 0)),
            pl.BlockSpec((H, H), lambda g, nb: (0, 0)),
            pl.BlockSpec((1, H), lambda g, nb: (0, 0)),
            pl.BlockSpec((H, OUT), lambda g, nb: (0, 0)),
            pl.BlockSpec((1, OUT), lambda g, nb: (0, 0)),
        ],
        out_specs=pl.BlockSpec((1, G, OUT), lambda g, nb: (g, 0, 0)),
        out_shape=jax.ShapeDtypeStruct((3, G, OUT), jnp.float32),
        scratch_shapes=[pltpu.VMEM((G, OUT), jnp.float32)],
    )(x3, aggr, aggr, batch3, scale, W_enc, b_enc, W1, b1, W2, b2, W_fc, b_fc)


def kernel(anchor_x, anchor_edge_index, anchor_edge_attr, anchor_batch,
           positive_x, positive_edge_index, positive_edge_attr, positive_batch,
           negative_x, negative_edge_index, negative_edge_attr, negative_batch,
           W_enc, b_enc, eps, W_edge, b_edge, W1, b1, W2, b2, W_fc, b_fc):
    x3 = jnp.stack([anchor_x[:, 0], positive_x[:, 0], negative_x[:, 0]])
    x3p = jnp.pad(x3, ((0, 0), (0, XP - N)))
    pe = E2 - E
    src3 = jnp.stack([anchor_edge_index[0], positive_edge_index[0],
                      negative_edge_index[0]])
    src3 = jnp.pad(src3, ((0, 0), (0, pe))).reshape(3, E2 // LN, LN)
    dpad = (N + (jnp.arange(pe, dtype=jnp.int32) % NPAD))[None, :]
    dst3 = jnp.stack([anchor_edge_index[1], positive_edge_index[1],
                      negative_edge_index[1]])
    dst3 = jnp.concatenate(
        [dst3, jnp.broadcast_to(dpad, (3, pe))], axis=1).reshape(3, E2 // LN, LN)
    ea3 = jnp.stack([anchor_edge_attr, positive_edge_attr, negative_edge_attr])
    ea0 = jnp.pad(ea3[:, :, 0], ((0, 0), (0, pe))).reshape(3, E2 // LN, LN)
    ea1 = jnp.pad(ea3[:, :, 1], ((0, 0), (0, pe))).reshape(3, E2 // LN, LN)
    params = jnp.concatenate([W_enc, W_edge, (b_enc + b_edge)[None, :]], axis=0)
    params = jnp.stack([params[:, :CH], params[:, CH:]])
    zeros = jnp.zeros((RPT, CH), jnp.float32)

    aggr = _edge_stage(x3p[0], x3p[1], x3p[2], src3, dst3, ea0, ea1,
                       params, zeros)

    batch3 = jnp.stack([anchor_batch, positive_batch,
                        negative_batch]).reshape(3 * NBLK, 1, BN)
    scale = (1.0 + eps).reshape(1, 1)
    out3 = _dense_stage(x3[:, :, None], aggr, batch3, scale,
                        W_enc, b_enc[None, :], W1, b1[None, :],
                        W2, b2[None, :], W_fc, b_fc[None, :])
    return (out3[0], out3[1], out3[2])


# speedup vs baseline: 6.5463x; 1.4523x over previous
import functools

import jax
import jax.numpy as jnp
from jax import lax
from jax.experimental import pallas as pl
from jax.experimental.pallas import tpu as pltpu
from jax.experimental.pallas import tpu_sc as plsc

N = 50000
E = 800000
H = 64
OUT = 64
G = 256

NC = 2
NS = 16
LN = 128
E2 = 819200
K = 512
NB = (E2 // NS) // K
KR = K // LN
CHUNK = 128
NCH = K // CHUNK
NGRP = K // 16
NPAD = 8
XP = 50048
RPT = 3128
RLAST = N - 15 * RPT
CH = 32


def _edge_kernel(xa_hbm, xb_hbm, xc_hbm, src_hbm, dst_hbm, ea0_hbm, ea1_hbm,
                 par_hbm, zer_hbm, aggr_hbm,
                 xs_v, src_v, dst_v, ea0_v, ea1_v, m_v, p_v, aggr_sh,
                 sem_in, sem_g, sem_s0, sem_s1):
    x_hbms = (xa_hbm, xb_hbm, xc_hbm)
    c = lax.axis_index("c")
    t = lax.axis_index("s")
    pltpu.sync_copy(par_hbm.at[c], p_v)
    u0 = p_v[0, 0:16]
    u1 = p_v[0, 16:32]
    v0 = p_v[1, 0:16]
    v1 = p_v[1, 16:32]
    w0 = p_v[2, 0:16]
    w1 = p_v[2, 16:32]
    b0 = p_v[3, 0:16]
    b1 = p_v[3, 16:32]
    hgrp = NGRP // 2

    for g in range(3):
        xg_hbm = x_hbms[g]

        @pl.when(t < NS - 1)
        def _():
            pltpu.sync_copy(zer_hbm, aggr_sh.at[pl.ds(t * RPT, RPT)])

        @pl.when(t == NS - 1)
        def _():
            pltpu.sync_copy(zer_hbm.at[pl.ds(0, RLAST + NPAD)],
                            aggr_sh.at[pl.ds(15 * RPT, RLAST + NPAD)])

        plsc.subcore_barrier()

        def issue_inputs(b, p):
            row = t * (NB * KR) + b * KR
            pltpu.async_copy(src_hbm.at[g, pl.ds(row, KR), :], src_v.at[p],
                             sem_in)
            pltpu.async_copy(ea0_hbm.at[g, pl.ds(row, KR), :], ea0_v.at[p],
                             sem_in)
            pltpu.async_copy(ea1_hbm.at[g, pl.ds(row, KR), :], ea1_v.at[p],
                             sem_in)
            pltpu.async_copy(dst_hbm.at[g, pl.ds(row, KR), :], dst_v.at[p],
                             sem_in)

        def wait_inputs(p):
            pltpu.make_async_copy(src_hbm.at[g, pl.ds(0, KR), :],
                                  src_v.at[p], sem_in).wait()
            pltpu.make_async_copy(ea0_hbm.at[g, pl.ds(0, KR), :],
                                  ea0_v.at[p], sem_in).wait()
            pltpu.make_async_copy(ea1_hbm.at[g, pl.ds(0, KR), :],
                                  ea1_v.at[p], sem_in).wait()
            pltpu.make_async_copy(dst_hbm.at[g, pl.ds(0, KR), :],
                                  dst_v.at[p], sem_in).wait()

        def drain_half(p, h):
            sm = sem_s0 if h == 0 else sem_s1
            for i in (2 * h, 2 * h + 1):
                pltpu.make_async_copy(m_v.at[pl.ds(i * CHUNK, CHUNK)],
                                      aggr_sh.at[dst_v.at[p, i]], sm).wait()

        def compute_half(p, h):
            def group_body(q, carry2):
                r = q >> 3
                o = (q & 7) * 16
                xs = xs_v[p, r, pl.ds(o, 16)]
                a0 = ea0_v[p, r, pl.ds(o, 16)]
                a1 = ea1_v[p, r, pl.ds(o, 16)]
                jb = q * 16
                for e in range(16):
                    xe = jnp.full((16,), xs[e])
                    ae = jnp.full((16,), a0[e])
                    ce = jnp.full((16,), a1[e])
                    m_v[jb + e, 0:16] = jnp.maximum(
                        xe * u0 + ae * v0 + ce * w0 + b0, 0.0)
                    m_v[jb + e, 16:32] = jnp.maximum(
                        xe * u1 + ae * v1 + ce * w1 + b1, 0.0)
                return carry2

            lax.fori_loop(h * hgrp, (h + 1) * hgrp, group_body, 0)
            sm = sem_s0 if h == 0 else sem_s1
            for i in (2 * h, 2 * h + 1):
                pltpu.async_copy(m_v.at[pl.ds(i * CHUNK, CHUNK)],
                                 aggr_sh.at[dst_v.at[p, i]], sm, add=True)

        def do_block(b, p, first):
            wait_inputs(p)
            for r in range(KR):
                pltpu.async_copy(xg_hbm.at[src_v.at[p, r]], xs_v.at[p, r],
                                 sem_g)

            @pl.when(b + 1 < NB)
            def _():
                issue_inputs(b + 1, (p + 1) & 3)

            for r in range(KR):
                pltpu.make_async_copy(xg_hbm.at[src_v.at[p, r]],
                                      xs_v.at[p, r], sem_g).wait()
            for h in range(2):
                if not first:
                    drain_half((p - 1) & 3, h)
                compute_half(p, h)

        issue_inputs(0, 0)
        for u in range(4):
            do_block(u, u, u == 0)

        def super_body(i, carry):
            for u in range(4):
                do_block(4 * i + u, u, False)
            return carry

        lax.fori_loop(1, NB // 4, super_body, 0)
        for h in range(2):
            drain_half(3, h)
        plsc.subcore_barrier()

        @pl.when(t < NS - 1)
        def _():
            pltpu.sync_copy(aggr_sh.at[pl.ds(t * RPT, RPT)],
                            aggr_hbm.at[g, c, pl.ds(t * RPT, RPT), :])

        @pl.when(t == NS - 1)
        def _():
            pltpu.sync_copy(aggr_sh.at[pl.ds(15 * RPT, RLAST)],
                            aggr_hbm.at[g, c, pl.ds(15 * RPT, RLAST), :])

        plsc.subcore_barrier()


def _edge_stage(xa, xb, xc, src3, dst3, ea0, ea1, params, zeros):
    mesh = plsc.VectorSubcoreMesh(core_axis_name="c", subcore_axis_name="s")
    return pl.kernel(
        _edge_kernel,
        mesh=mesh,
        compiler_params=pltpu.CompilerParams(use_tc_tiling_on_sc=False),
        out_type=jax.ShapeDtypeStruct((3, NC, N, CH), jnp.float32),
        scratch_types=[
            pltpu.VMEM((4, KR, LN), jnp.float32),
            pltpu.VMEM((4, KR, LN), jnp.int32),
            pltpu.VMEM((4, KR, LN), jnp.int32),
            pltpu.VMEM((4, KR, LN), jnp.float32),
            pltpu.VMEM((4, KR, LN), jnp.float32),
            pltpu.VMEM((K, CH), jnp.float32),
            pltpu.VMEM((4, CH), jnp.float32),
            pltpu.VMEM_SHARED((N + NPAD, CH), jnp.float32),
            pltpu.SemaphoreType.DMA,
            pltpu.SemaphoreType.DMA,
            pltpu.SemaphoreType.DMA,
            pltpu.SemaphoreType.DMA,
        ],
    )(xa, xb, xc, src3, dst3, ea0, ea1, params, zeros)


BN = 2000
NBLK = N // BN


def _dense_kernel(x_ref, alo_ref, ahi_ref, batch_ref, scale_ref, wenc_ref,
                  benc_ref, w1_ref, b1_ref, w2_ref, b2_ref, wfc_ref, bfc_ref,
                  out_ref, pooled_scr):
    nb = pl.program_id(1)

    @pl.when(nb == 0)
    def _():
        pooled_scr[...] = jnp.zeros_like(pooled_scr)

    xcol = x_ref[0]
    h = xcol * wenc_ref[...] + benc_ref[...]
    aggr = jnp.concatenate([alo_ref[0, 0], ahi_ref[0, 0]], axis=-1)
    z = scale_ref[0, 0] * h + aggr
    z = jnp.maximum(jnp.dot(z, w1_ref[...], preferred_element_type=jnp.float32)
                    + b1_ref[...], 0.0)
    z = jnp.maximum(jnp.dot(z, w2_ref[...], preferred_element_type=jnp.float32)
                    + b2_ref[...], 0.0)
    seg = batch_ref[0, 0, :]
    onehot = (lax.broadcasted_iota(jnp.int32, (G, BN), 0) ==
              seg[None, :]).astype(jnp.float32)
    pooled_scr[...] += jnp.dot(onehot, z, preferred_element_type=jnp.float32)

    @pl.when(nb == NBLK - 1)
    def _():
        out_ref[0] = (jnp.dot(pooled_scr[...], wfc_ref[...],
                              preferred_element_type=jnp.float32)
                      + bfc_ref[...])


def _dense_stage(x3, aggr, batch3, scale, W_enc, b_enc, W1, b1, W2, b2,
                 W_fc, b_fc):
    return pl.pallas_call(
        _dense_kernel,
        grid=(3, NBLK),
        in_specs=[
            pl.BlockSpec((1, BN, 1), lambda g, nb: (g, nb, 0)),
            pl.BlockSpec((1, 1, BN, CH), lambda g, nb: (g, 0, nb, 0)),
            pl.BlockSpec((1, 1, BN, CH), lambda g, nb: (g, 1, nb, 0)),
            pl.BlockSpec((1, 1, BN), lambda g, nb: (g * NBLK + nb, 0, 0)),
            pl.BlockSpec((1, 1), lambda g, nb: (0, 0)),
            pl.BlockSpec((1, H), lambda g, nb: (0, 0)),
            pl.BlockSpec((1, H), lambda g, nb: (0, 0)),
            pl.BlockSpec((H, H), lambda g, nb: (0, 0)),
            pl.BlockSpec((1, H), lambda g, nb: (0, 0)),
            pl.BlockSpec((H, H), lambda g, nb: (0, 0)),
            pl.BlockSpec((1, H), lambda g, nb: (0, 0)),
            pl.BlockSpec((H, OUT), lambda g, nb: (0, 0)),
            pl.BlockSpec((1, OUT), lambda g, nb: (0, 0)),
        ],
        out_specs=pl.BlockSpec((1, G, OUT), lambda g, nb: (g, 0, 0)),
        out_shape=jax.ShapeDtypeStruct((3, G, OUT), jnp.float32),
        scratch_shapes=[pltpu.VMEM((G, OUT), jnp.float32)],
    )(x3, aggr, aggr, batch3, scale, W_enc, b_enc, W1, b1, W2, b2, W_fc, b_fc)


def kernel(anchor_x, anchor_edge_index, anchor_edge_attr, anchor_batch,
           positive_x, positive_edge_index, positive_edge_attr, positive_batch,
           negative_x, negative_edge_index, negative_edge_attr, negative_batch,
           W_enc, b_enc, eps, W_edge, b_edge, W1, b1, W2, b2, W_fc, b_fc):
    x3 = jnp.stack([anchor_x[:, 0], positive_x[:, 0], negative_x[:, 0]])
    x3p = jnp.pad(x3, ((0, 0), (0, XP - N)))
    pe = E2 - E
    src3 = jnp.stack([anchor_edge_index[0], positive_edge_index[0],
                      negative_edge_index[0]])
    src3 = jnp.pad(src3, ((0, 0), (0, pe))).reshape(3, E2 // LN, LN)
    dpad = (N + (jnp.arange(pe, dtype=jnp.int32) % NPAD))[None, :]
    dst3 = jnp.stack([anchor_edge_index[1], positive_edge_index[1],
                      negative_edge_index[1]])
    dst3 = jnp.concatenate(
        [dst3, jnp.broadcast_to(dpad, (3, pe))], axis=1).reshape(3, E2 // LN, LN)
    ea3 = jnp.stack([anchor_edge_attr, positive_edge_attr, negative_edge_attr])
    ea0 = jnp.pad(ea3[:, :, 0], ((0, 0), (0, pe))).reshape(3, E2 // LN, LN)
    ea1 = jnp.pad(ea3[:, :, 1], ((0, 0), (0, pe))).reshape(3, E2 // LN, LN)
    params = jnp.concatenate([W_enc, W_edge, (b_enc + b_edge)[None, :]], axis=0)
    params = jnp.stack([params[:, :CH], params[:, CH:]])
    zeros = jnp.zeros((RPT, CH), jnp.float32)

    aggr = _edge_stage(x3p[0], x3p[1], x3p[2], src3, dst3, ea0, ea1,
                       params, zeros)

    batch3 = jnp.stack([anchor_batch, positive_batch,
                        negative_batch]).reshape(3 * NBLK, 1, BN)
    scale = (1.0 + eps).reshape(1, 1)
    out3 = _dense_stage(x3[:, :, None], aggr, batch3, scale,
                        W_enc, b_enc[None, :], W1, b1[None, :],
                        W2, b2[None, :], W_fc, b_fc[None, :])
    return (out3[0], out3[1], out3[2])


# speedup vs baseline: 7.6410x; 1.1672x over previous
import functools

import jax
import jax.numpy as jnp
from jax import lax
from jax.experimental import pallas as pl
from jax.experimental.pallas import tpu as pltpu
from jax.experimental.pallas import tpu_sc as plsc

N = 50000
E = 800000
H = 64
OUT = 64
G = 256

NC = 2
NS = 16
LN = 128
E2 = 819200
K = 512
NB = (E2 // NS) // K
KR = K // LN
CHUNK = 128
NCH = K // CHUNK
NGRP = K // 16
NPAD = 8
XP = 50048
RPT = 3128
RLAST = N - 15 * RPT
CH = 32


def _edge_kernel(xa_hbm, xb_hbm, xc_hbm, src_hbm, dst_hbm, ea0_hbm, ea1_hbm,
                 par_hbm, zer_hbm, aggr_hbm,
                 xs_v, src_v, dst_v, ea0_v, ea1_v, m_v, p_v, aggr_sh,
                 sem_in, sem_ga, sem_gb, sem_s0, sem_s1):
    x_hbms = (xa_hbm, xb_hbm, xc_hbm)
    c = lax.axis_index("c")
    t = lax.axis_index("s")
    pltpu.sync_copy(par_hbm.at[c], p_v)
    u0 = p_v[0, 0:16]
    u1 = p_v[0, 16:32]
    v0 = p_v[1, 0:16]
    v1 = p_v[1, 16:32]
    w0 = p_v[2, 0:16]
    w1 = p_v[2, 16:32]
    b0 = p_v[3, 0:16]
    b1 = p_v[3, 16:32]
    hgrp = NGRP // 2

    for g in range(3):
        xg_hbm = x_hbms[g]

        @pl.when(t < NS - 1)
        def _():
            pltpu.sync_copy(zer_hbm, aggr_sh.at[pl.ds(t * RPT, RPT)])

        @pl.when(t == NS - 1)
        def _():
            pltpu.sync_copy(zer_hbm.at[pl.ds(0, RLAST + NPAD)],
                            aggr_sh.at[pl.ds(15 * RPT, RLAST + NPAD)])

        plsc.subcore_barrier()

        def issue_inputs(b, p):
            row = t * (NB * KR) + b * KR
            pltpu.async_copy(src_hbm.at[g, pl.ds(row, KR), :], src_v.at[p],
                             sem_in)
            pltpu.async_copy(ea0_hbm.at[g, pl.ds(row, KR), :], ea0_v.at[p],
                             sem_in)
            pltpu.async_copy(ea1_hbm.at[g, pl.ds(row, KR), :], ea1_v.at[p],
                             sem_in)
            pltpu.async_copy(dst_hbm.at[g, pl.ds(row, KR), :], dst_v.at[p],
                             sem_in)

        def wait_inputs(p):
            pltpu.make_async_copy(src_hbm.at[g, pl.ds(0, KR), :],
                                  src_v.at[p], sem_in).wait()
            pltpu.make_async_copy(ea0_hbm.at[g, pl.ds(0, KR), :],
                                  ea0_v.at[p], sem_in).wait()
            pltpu.make_async_copy(ea1_hbm.at[g, pl.ds(0, KR), :],
                                  ea1_v.at[p], sem_in).wait()
            pltpu.make_async_copy(dst_hbm.at[g, pl.ds(0, KR), :],
                                  dst_v.at[p], sem_in).wait()

        def drain_half(p, h):
            sm = sem_s0 if h == 0 else sem_s1
            for i in (2 * h, 2 * h + 1):
                pltpu.make_async_copy(m_v.at[pl.ds(i * CHUNK, CHUNK)],
                                      aggr_sh.at[dst_v.at[p, i]], sm).wait()

        def compute_half(p, h):
            def group_body(q, carry2):
                r = q >> 3
                o = (q & 7) * 16
                xs = xs_v[p, r, pl.ds(o, 16)]
                a0 = ea0_v[p, r, pl.ds(o, 16)]
                a1 = ea1_v[p, r, pl.ds(o, 16)]
                jb = q * 16
                for e in range(16):
                    xe = jnp.full((16,), xs[e])
                    ae = jnp.full((16,), a0[e])
                    ce = jnp.full((16,), a1[e])
                    m_v[jb + e, 0:16] = jnp.maximum(
                        xe * u0 + ae * v0 + ce * w0 + b0, 0.0)
                    m_v[jb + e, 16:32] = jnp.maximum(
                        xe * u1 + ae * v1 + ce * w1 + b1, 0.0)
                return carry2

            lax.fori_loop(h * hgrp, (h + 1) * hgrp, group_body, 0)
            sm = sem_s0 if h == 0 else sem_s1
            for i in (2 * h, 2 * h + 1):
                pltpu.async_copy(m_v.at[pl.ds(i * CHUNK, CHUNK)],
                                 aggr_sh.at[dst_v.at[p, i]], sm, add=True)

        def issue_gathers(p, sm):
            for r in range(KR):
                pltpu.async_copy(xg_hbm.at[src_v.at[p, r]], xs_v.at[p, r], sm)

        def wait_gathers(p, sm):
            for r in range(KR):
                pltpu.make_async_copy(xg_hbm.at[src_v.at[p, r]],
                                      xs_v.at[p, r], sm).wait()

        def do_block(b, u, first):
            pn = (u + 1) & 3
            gsem_cur = sem_ga if (u & 1) == 0 else sem_gb
            gsem_nxt = sem_gb if (u & 1) == 0 else sem_ga

            @pl.when(b + 1 < NB)
            def _():
                wait_inputs(pn)
                issue_gathers(pn, gsem_nxt)

                @pl.when(b + 2 < NB)
                def _():
                    issue_inputs(b + 2, (u + 2) & 3)

            wait_gathers(u, gsem_cur)
            for h in range(2):
                if not first:
                    drain_half((u - 1) & 3, h)
                compute_half(u, h)

        issue_inputs(0, 0)
        wait_inputs(0)
        issue_gathers(0, sem_ga)
        issue_inputs(1, 1)
        for u in range(4):
            do_block(u, u, u == 0)

        def super_body(i, carry):
            for u in range(4):
                do_block(4 * i + u, u, False)
            return carry

        lax.fori_loop(1, NB // 4, super_body, 0)
        for h in range(2):
            drain_half(3, h)
        plsc.subcore_barrier()

        @pl.when(t < NS - 1)
        def _():
            pltpu.sync_copy(aggr_sh.at[pl.ds(t * RPT, RPT)],
                            aggr_hbm.at[g, c, pl.ds(t * RPT, RPT), :])

        @pl.when(t == NS - 1)
        def _():
            pltpu.sync_copy(aggr_sh.at[pl.ds(15 * RPT, RLAST)],
                            aggr_hbm.at[g, c, pl.ds(15 * RPT, RLAST), :])

        plsc.subcore_barrier()


def _edge_stage(xa, xb, xc, src3, dst3, ea0, ea1, params, zeros):
    mesh = plsc.VectorSubcoreMesh(core_axis_name="c", subcore_axis_name="s")
    return pl.kernel(
        _edge_kernel,
        mesh=mesh,
        compiler_params=pltpu.CompilerParams(use_tc_tiling_on_sc=False),
        out_type=jax.ShapeDtypeStruct((3, NC, N, CH), jnp.float32),
        scratch_types=[
            pltpu.VMEM((4, KR, LN), jnp.float32),
            pltpu.VMEM((4, KR, LN), jnp.int32),
            pltpu.VMEM((4, KR, LN), jnp.int32),
            pltpu.VMEM((4, KR, LN), jnp.float32),
            pltpu.VMEM((4, KR, LN), jnp.float32),
            pltpu.VMEM((K, CH), jnp.float32),
            pltpu.VMEM((4, CH), jnp.float32),
            pltpu.VMEM_SHARED((N + NPAD, CH), jnp.float32),
            pltpu.SemaphoreType.DMA,
            pltpu.SemaphoreType.DMA,
            pltpu.SemaphoreType.DMA,
            pltpu.SemaphoreType.DMA,
            pltpu.SemaphoreType.DMA,
        ],
    )(xa, xb, xc, src3, dst3, ea0, ea1, params, zeros)


BN = 2000
NBLK = N // BN


def _dense_kernel(x_ref, alo_ref, ahi_ref, batch_ref, scale_ref, wenc_ref,
                  benc_ref, w1_ref, b1_ref, w2_ref, b2_ref, wfc_ref, bfc_ref,
                  out_ref, pooled_scr):
    nb = pl.program_id(1)

    @pl.when(nb == 0)
    def _():
        pooled_scr[...] = jnp.zeros_like(pooled_scr)

    xcol = x_ref[0]
    h = xcol * wenc_ref[...] + benc_ref[...]
    aggr = jnp.concatenate([alo_ref[0, 0], ahi_ref[0, 0]], axis=-1)
    z = scale_ref[0, 0] * h + aggr
    z = jnp.maximum(jnp.dot(z, w1_ref[...], preferred_element_type=jnp.float32)
                    + b1_ref[...], 0.0)
    z = jnp.maximum(jnp.dot(z, w2_ref[...], preferred_element_type=jnp.float32)
                    + b2_ref[...], 0.0)
    seg = batch_ref[0, 0, :]
    onehot = (lax.broadcasted_iota(jnp.int32, (G, BN), 0) ==
              seg[None, :]).astype(jnp.float32)
    pooled_scr[...] += jnp.dot(onehot, z, preferred_element_type=jnp.float32)

    @pl.when(nb == NBLK - 1)
    def _():
        out_ref[0] = (jnp.dot(pooled_scr[...], wfc_ref[...],
                              preferred_element_type=jnp.float32)
                      + bfc_ref[...])


def _dense_stage(x3, aggr, batch3, scale, W_enc, b_enc, W1, b1, W2, b2,
                 W_fc, b_fc):
    return pl.pallas_call(
        _dense_kernel,
        grid=(3, NBLK),
        in_specs=[
            pl.BlockSpec((1, BN, 1), lambda g, nb: (g, nb, 0)),
            pl.BlockSpec((1, 1, BN, CH), lambda g, nb: (g, 0, nb, 0)),
            pl.BlockSpec((1, 1, BN, CH), lambda g, nb: (g, 1, nb, 0)),
            pl.BlockSpec((1, 1, BN), lambda g, nb: (g * NBLK + nb, 0, 0)),
            pl.BlockSpec((1, 1), lambda g, nb: (0, 0)),
            pl.BlockSpec((1, H), lambda g, nb: (0, 0)),
            pl.BlockSpec((1, H), lambda g, nb: (0, 0)),
            pl.BlockSpec((H, H), lambda g, nb: (0, 0)),
            pl.BlockSpec((1, H), lambda g, nb: (0, 0)),
            pl.BlockSpec((H, H), lambda g, nb: (0, 0)),
            pl.BlockSpec((1, H), lambda g, nb: (0, 0)),
            pl.BlockSpec((H, OUT), lambda g, nb: (0, 0)),
            pl.BlockSpec((1, OUT), lambda g, nb: (0, 0)),
        ],
        out_specs=pl.BlockSpec((1, G, OUT), lambda g, nb: (g, 0, 0)),
        out_shape=jax.ShapeDtypeStruct((3, G, OUT), jnp.float32),
        scratch_shapes=[pltpu.VMEM((G, OUT), jnp.float32)],
    )(x3, aggr, aggr, batch3, scale, W_enc, b_enc, W1, b1, W2, b2, W_fc, b_fc)


def kernel(anchor_x, anchor_edge_index, anchor_edge_attr, anchor_batch,
           positive_x, positive_edge_index, positive_edge_attr, positive_batch,
           negative_x, negative_edge_index, negative_edge_attr, negative_batch,
           W_enc, b_enc, eps, W_edge, b_edge, W1, b1, W2, b2, W_fc, b_fc):
    x3 = jnp.stack([anchor_x[:, 0], positive_x[:, 0], negative_x[:, 0]])
    x3p = jnp.pad(x3, ((0, 0), (0, XP - N)))
    pe = E2 - E
    src3 = jnp.stack([anchor_edge_index[0], positive_edge_index[0],
                      negative_edge_index[0]])
    src3 = jnp.pad(src3, ((0, 0), (0, pe))).reshape(3, E2 // LN, LN)
    dpad = (N + (jnp.arange(pe, dtype=jnp.int32) % NPAD))[None, :]
    dst3 = jnp.stack([anchor_edge_index[1], positive_edge_index[1],
                      negative_edge_index[1]])
    dst3 = jnp.concatenate(
        [dst3, jnp.broadcast_to(dpad, (3, pe))], axis=1).reshape(3, E2 // LN, LN)
    ea3 = jnp.stack([anchor_edge_attr, positive_edge_attr, negative_edge_attr])
    ea0 = jnp.pad(ea3[:, :, 0], ((0, 0), (0, pe))).reshape(3, E2 // LN, LN)
    ea1 = jnp.pad(ea3[:, :, 1], ((0, 0), (0, pe))).reshape(3, E2 // LN, LN)
    params = jnp.concatenate([W_enc, W_edge, (b_enc + b_edge)[None, :]], axis=0)
    params = jnp.stack([params[:, :CH], params[:, CH:]])
    zeros = jnp.zeros((RPT, CH), jnp.float32)

    aggr = _edge_stage(x3p[0], x3p[1], x3p[2], src3, dst3, ea0, ea1,
                       params, zeros)

    batch3 = jnp.stack([anchor_batch, positive_batch,
                        negative_batch]).reshape(3 * NBLK, 1, BN)
    scale = (1.0 + eps).reshape(1, 1)
    out3 = _dense_stage(x3[:, :, None], aggr, batch3, scale,
                        W_enc, b_enc[None, :], W1, b1[None, :],
                        W2, b2[None, :], W_fc, b_fc[None, :])
    return (out3[0], out3[1], out3[2])
